# Initial kernel scaffold; baseline (speedup 1.0000x reference)
import jax
import jax.numpy as jnp
from jax.experimental import pallas as pl


def kernel(input_ids, vision_features, W):
    raise NotImplementedError("write your pallas kernel here")


# speedup vs baseline: 1.6318x; 1.6318x over previous
import functools

import jax
import jax.numpy as jnp
from jax import lax
from jax.experimental import pallas as pl
from jax.experimental.pallas import tpu as pltpu
from jax.experimental.pallas import tpu_sc as plsc

_IMAGE_TOKEN_ID = 31999
_NUM_PATCHES = 256
_NC = 2
_NS = 16
_CH = 16


@functools.partial(jax.jit, static_argnames=("B", "S", "D"))
def _sc_embed(ids_flat, vis_flat, table, *, B, S, D):
    NW = _NC * _NS
    BT = B * S
    TOK = BT // NW
    WPB = NW // B
    NCH = TOK // _CH
    VR = (B * _NUM_PATCHES) // NW

    mesh = plsc.VectorSubcoreMesh(core_axis_name="c", subcore_axis_name="s")

    @functools.partial(
        pl.kernel,
        mesh=mesh,
        out_type=jax.ShapeDtypeStruct((BT, D), jnp.float32),
        scratch_types=[
            pltpu.VMEM((S,), jnp.int32),
            pltpu.VMEM((2, _CH, D), jnp.float32),
            pltpu.SemaphoreType.DMA,
            pltpu.SemaphoreType.DMA,
        ],
    )
    def run(ids_hbm, vis_hbm, w_hbm, out_hbm, ids_v, buf, sem0, sem1):
        c = lax.axis_index("c")
        s = lax.axis_index("s")
        w = c * _NS + s
        b = w // WPB
        col0 = (w % WPB) * TOK

        pltpu.sync_copy(ids_hbm.at[pl.ds(b * S, S)], ids_v)

        iota = lax.iota(jnp.int32, 16)

        def scan_body(j, acc):
            v = ids_v[pl.ds(j * 16, 16)]
            pos = jnp.where(v == _IMAGE_TOKEN_ID, iota + j * 16, S)
            return jnp.minimum(acc, pos)

        acc = lax.fori_loop(
            0, S // 16, scan_body, jnp.full((16,), S, jnp.int32)
        )
        def shuffle(x, idx):
            return lax.gather(
                x,
                idx[:, None],
                lax.GatherDimensionNumbers(
                    offset_dims=(),
                    collapsed_slice_dims=(0,),
                    start_index_map=(0,),
                ),
                slice_sizes=(1,),
                mode=lax.GatherScatterMode.PROMISE_IN_BOUNDS,
            )

        for k in (1, 2, 4, 8):
            acc = jnp.minimum(acc, shuffle(acc, iota ^ k))
        start_splat = acc

        sems = (sem0, sem1)

        def gather_start(i, slot):
            idx = ids_v.at[pl.ds(col0 + i * _CH, _CH)]
            return pltpu.async_copy(w_hbm.at[idx], buf.at[slot], sems[slot])

        copies = [gather_start(0, 0), None]
        for i in range(NCH):
            slot = i % 2
            if i + 1 < NCH:
                copies[(i + 1) % 2] = gather_start(i + 1, (i + 1) % 2)
            copies[slot].wait()
            pltpu.sync_copy(
                buf.at[slot], out_hbm.at[pl.ds(w * TOK + i * _CH, _CH)]
            )

        plsc.subcore_barrier()

        p0 = (w % WPB) * VR
        for q in range(VR // _CH):
            pltpu.sync_copy(
                vis_hbm.at[pl.ds(w * VR + q * _CH, _CH)], buf.at[q % 2]
            )
            dst_idx = b * S + start_splat + p0 + q * _CH + iota
            pltpu.sync_copy(buf.at[q % 2], out_hbm.at[dst_idx])

    return run(ids_flat, vis_flat, table)


def kernel(input_ids, vision_features, W):
    B, S = input_ids.shape
    _, D = W.shape
    ids_flat = input_ids.reshape(B * S).astype(jnp.int32)
    vis_flat = vision_features.reshape(B * _NUM_PATCHES, D).astype(jnp.float32)
    out = _sc_embed(ids_flat, vis_flat, W.astype(jnp.float32), B=B, S=S, D=D)
    return out.reshape(B, S, D)
